# baseline (speedup 1.0000x reference)
import functools

import jax
import jax.numpy as jnp
from jax import lax
from jax.experimental import pallas as pl
from jax.experimental.pallas import tpu as pltpu
from jax.experimental.pallas import tpu_sc as plsc

_NC, _NS = 2, 16
_NW = _NC * _NS
_NIDS = 12800
_PERW = _NIDS // _NW
_CHUNKS = ((0, 128), (128, 128), (256, 128), (384, 16))
_SREG = 16
_W1 = 8448
_BM = 256


def _sc_gather_body(w_hbm, b16_hbm, ids_hbm, out_w_hbm, out_b_hbm,
                    ids_v, rows_v, lanes_v, w_v, brow_v, bout_v, sem_w, sem_b):
    wid = lax.axis_index("s") * _NC + lax.axis_index("c")
    base = wid * _PERW
    pltpu.sync_copy(ids_hbm.at[pl.ds(base, _PERW)], ids_v)

    @pl.loop(0, _PERW, step=_SREG)
    def _(i):
        c = ids_v[pl.ds(i, _SREG)]
        rows_v[pl.ds(i, _SREG)] = c >> 4
        lanes_v[pl.ds(i, _SREG)] = c & 15

    copies = []
    for off, n in _CHUNKS:
        copies.append(pltpu.async_copy(
            w_hbm.at[ids_v.at[pl.ds(off, n)]], w_v.at[pl.ds(off, n)], sem_w))
        copies.append(pltpu.async_copy(
            b16_hbm.at[rows_v.at[pl.ds(off, n)]], brow_v.at[pl.ds(off, n)], sem_b))
    for cp in copies:
        cp.wait()

    @pl.loop(0, _PERW, step=_SREG)
    def _(i):
        idx0 = lax.iota(jnp.int32, _SREG) + i
        bout_v[pl.ds(i, _SREG)] = plsc.load_gather(
            brow_v, [idx0, lanes_v[pl.ds(i, _SREG)]])

    pltpu.sync_copy(w_v, out_w_hbm.at[pl.ds(base, _PERW)])
    pltpu.sync_copy(bout_v, out_b_hbm.at[pl.ds(base, _PERW)])


def _sc_gather(softmax_w, b16, ids_all):
    kern = pl.kernel(
        _sc_gather_body,
        out_type=[jax.ShapeDtypeStruct((_NIDS, 64), jnp.float32),
                  jax.ShapeDtypeStruct((_NIDS,), jnp.float32)],
        mesh=plsc.VectorSubcoreMesh(core_axis_name="c", subcore_axis_name="s",
                                    num_cores=_NC, num_subcores=_NS),
        scratch_types=[
            pltpu.VMEM((_PERW,), jnp.int32),
            pltpu.VMEM((_PERW,), jnp.int32),
            pltpu.VMEM((_PERW,), jnp.int32),
            pltpu.VMEM((_PERW, 64), jnp.float32),
            pltpu.VMEM((_PERW, 16), jnp.float32),
            pltpu.VMEM((_PERW,), jnp.float32),
            pltpu.SemaphoreType.DMA,
            pltpu.SemaphoreType.DMA,
        ],
        compiler_params=pltpu.CompilerParams(needs_layout_passes=False,
                                             use_tc_tiling_on_sc=False),
    )
    return kern(softmax_w, b16, ids_all)


def _tc_body(x_ref, w_ref, tw_ref, bs_ref, bt_ref, lbl_ref, ids_ref, sf_ref,
             tf_ref, out_ref):
    x = x_ref[...]
    m = lax.dot_general(x, w_ref[...], (((1,), (1,)), ((), ())),
                        preferred_element_type=jnp.float32)
    m = jnp.where(lbl_ref[...] == ids_ref[...], jnp.float32(-1e37), m)
    m = m + (bs_ref[...] - jnp.log(sf_ref[...]))
    t = (jnp.sum(x * tw_ref[...], axis=1, keepdims=True)
         + bt_ref[...] - jnp.log(tf_ref[...]))
    col = lax.broadcasted_iota(jnp.int32, (_BM, 8193), 1)
    out_ref[...] = jnp.where(col == 0, t, m[:, :8193])


def _tc_logits(inputs, gw, gb, labels, ids_mask, sf_pad, true_freq):
    B = inputs.shape[0]
    return pl.pallas_call(
        _tc_body,
        grid=(B // _BM,),
        in_specs=[
            pl.BlockSpec((_BM, 64), lambda i: (i, 0)),
            pl.BlockSpec((_W1, 64), lambda i: (0, 0)),
            pl.BlockSpec((_BM, 64), lambda i: (i + _W1 // _BM, 0)),
            pl.BlockSpec((1, _W1), lambda i: (0, 0)),
            pl.BlockSpec((_BM, 1), lambda i: (i + _W1 // _BM, 0)),
            pl.BlockSpec((_BM, 1), lambda i: (i, 0)),
            pl.BlockSpec((1, _W1), lambda i: (0, 0)),
            pl.BlockSpec((1, _W1), lambda i: (0, 0)),
            pl.BlockSpec((_BM, 1), lambda i: (i, 0)),
        ],
        out_specs=pl.BlockSpec((_BM, 8193), lambda i: (i, 0)),
        out_shape=jax.ShapeDtypeStruct((B, 8193), jnp.float32),
    )(inputs, gw, gw, gb.reshape(1, _NIDS), gb.reshape(_NIDS, 1),
      labels.reshape(B, 1), ids_mask, sf_pad, true_freq.reshape(B, 1))


def kernel(inputs, labels, softmax_w, softmax_b, sample_ids, true_freq,
           sample_freq):
    B = inputs.shape[0]
    S = sample_ids.shape[0]
    z1 = jnp.zeros((1,), jnp.int32)
    ids_all = jnp.concatenate([
        z1, sample_ids, jnp.zeros((_W1 - S - 1,), jnp.int32),
        labels, jnp.zeros((_NIDS - _W1 - B,), jnp.int32)])
    b16 = softmax_b.reshape(-1, _SREG)
    gw, gb = _sc_gather(softmax_w, b16, ids_all)

    neg1 = jnp.full((1,), -1, jnp.int32)
    ids_mask = jnp.concatenate(
        [neg1, sample_ids, jnp.full((_W1 - S - 1,), -1, jnp.int32)]
    ).reshape(1, _W1)
    one1 = jnp.ones((1,), jnp.float32)
    sf_pad = jnp.concatenate(
        [one1, sample_freq, jnp.ones((_W1 - S - 1,), jnp.float32)]
    ).reshape(1, _W1)

    logits = _tc_logits(inputs, gw, gb, labels, ids_mask, sf_pad, true_freq)
    return logits, jnp.zeros((B,), jnp.int32)


# speedup vs baseline: 1.1109x; 1.1109x over previous
import jax
import jax.numpy as jnp
from jax import lax
from jax.experimental import pallas as pl
from jax.experimental.pallas import tpu as pltpu
from jax.experimental.pallas import tpu_sc as plsc

_NC, _NS = 2, 16
_NW = _NC * _NS
_NIDS = 12800
_PERW = _NIDS // _NW
_CHUNKS = ((0, 128), (128, 128), (256, 128), (384, 16))
_SREG = 16
_W1 = 8448
_BM = 256
_NTOK = 1000000
_OWN = _NTOK // _NS
_HALF = _OWN // 2
_STAGE = 31264


def _sc_gather_body(w_hbm, b_hbm, ids_hbm, out_w_hbm, out_b_hbm,
                    allids_v, w_v, bchunk_v, full_v, sum_v, acc_v,
                    shared_b, sem_w, sem_b):
    cid = lax.axis_index("c")
    sid = lax.axis_index("s")
    wid = sid * _NC + cid
    base = wid * _PERW
    pltpu.sync_copy(ids_hbm, allids_v)

    copies = []
    for off, n in _CHUNKS:
        copies.append(pltpu.async_copy(
            w_hbm.at[allids_v.at[pl.ds(base + off, n)]],
            w_v.at[pl.ds(off, n)], sem_w))

    own_lo = sid * _OWN
    for p in range(2):
        p_lo = own_lo + p * _HALF
        stage_lo = (p_lo // _SREG) * _SREG
        pltpu.async_copy(b_hbm.at[pl.ds(stage_lo, _STAGE)], bchunk_v,
                         sem_b).wait()

        @pl.loop(0, _NIDS, step=_SREG)
        def _(i):
            ids16 = allids_v[pl.ds(i, _SREG)]
            own = (ids16 >= p_lo) & (ids16 < p_lo + _HALF)
            loc = ids16 - stage_lo
            vals = plsc.load_gather(bchunk_v, [loc], mask=own)
            vals = jnp.where(own, vals, jnp.float32(0.0))
            if p == 0:
                full_v[pl.ds(i, _SREG)] = vals
            else:
                full_v[pl.ds(i, _SREG)] = full_v[pl.ds(i, _SREG)] + vals

    pltpu.sync_copy(full_v, shared_b.at[sid])
    plsc.subcore_barrier()
    slot = cid * (_NIDS // _NC) + sid * _PERW
    pltpu.sync_copy(shared_b.at[pl.ds(0, _NS), pl.ds(slot, _PERW)], sum_v)

    @pl.loop(0, _PERW, step=_SREG)
    def _(i):
        s = sum_v.at[0][pl.ds(i, _SREG)]
        for r in range(1, _NS):
            s = s + sum_v.at[r][pl.ds(i, _SREG)]
        acc_v[pl.ds(i, _SREG)] = s

    pltpu.sync_copy(acc_v, out_b_hbm.at[pl.ds(slot, _PERW)])

    for cp in copies:
        cp.wait()
    pltpu.sync_copy(w_v, out_w_hbm.at[pl.ds(base, _PERW)])


def _sc_gather(softmax_w, softmax_b, ids_all):
    kern = pl.kernel(
        _sc_gather_body,
        out_type=[jax.ShapeDtypeStruct((_NIDS, 64), jnp.float32),
                  jax.ShapeDtypeStruct((_NIDS,), jnp.float32)],
        mesh=plsc.VectorSubcoreMesh(core_axis_name="c", subcore_axis_name="s",
                                    num_cores=_NC, num_subcores=_NS),
        scratch_types=[
            pltpu.VMEM((_NIDS,), jnp.int32),
            pltpu.VMEM((_PERW, 64), jnp.float32),
            pltpu.VMEM((_STAGE,), jnp.float32),
            pltpu.VMEM((_NIDS,), jnp.float32),
            pltpu.VMEM((_NS, _PERW), jnp.float32),
            pltpu.VMEM((_PERW,), jnp.float32),
            pltpu.VMEM_SHARED((_NS, _NIDS), jnp.float32),
            pltpu.SemaphoreType.DMA,
            pltpu.SemaphoreType.DMA,
        ],
        compiler_params=pltpu.CompilerParams(needs_layout_passes=False,
                                             use_tc_tiling_on_sc=False),
    )
    return kern(softmax_w, softmax_b, ids_all)


def _tc_body(xT_ref, w_ref, tw_ref, bs_ref, bt_ref, lbl_ref, ids_ref, sf_ref,
             tf_ref, out_ref):
    xT = xT_ref[...]
    mT = lax.dot_general(w_ref[...], xT, (((1,), (0,)), ((), ())),
                         preferred_element_type=jnp.float32)
    acc = ids_ref[...].T == lbl_ref[...]
    mT = jnp.where(acc, jnp.float32(-1e37), mT)
    mT = mT + (bs_ref[...] - jnp.log(sf_ref[...])).T
    twT = tw_ref[...].T
    t_row = (jnp.sum(xT * twT, axis=0, keepdims=True)
             + bt_ref[...] - jnp.log(tf_ref[...]))
    row = lax.broadcasted_iota(jnp.int32, (8193, _BM), 0)
    out_ref[...] = jnp.where(row == 0, t_row, mT[:8193, :])


def _tc_logits(xT, gw, gb_row, lbl_row, ids_row, sf_row, tf_row):
    B = xT.shape[1]
    return pl.pallas_call(
        _tc_body,
        grid=(B // _BM,),
        in_specs=[
            pl.BlockSpec((64, _BM), lambda i: (0, i)),
            pl.BlockSpec((_W1, 64), lambda i: (0, 0)),
            pl.BlockSpec((_BM, 64), lambda i: (i + _W1 // _BM, 0)),
            pl.BlockSpec((1, _W1), lambda i: (0, 0)),
            pl.BlockSpec((1, _BM), lambda i: (0, i + _W1 // _BM)),
            pl.BlockSpec((1, _BM), lambda i: (0, i)),
            pl.BlockSpec((1, _W1), lambda i: (0, 0)),
            pl.BlockSpec((1, _W1), lambda i: (0, 0)),
            pl.BlockSpec((1, _BM), lambda i: (0, i)),
        ],
        out_specs=pl.BlockSpec((8193, _BM), lambda i: (0, i)),
        out_shape=jax.ShapeDtypeStruct((8193, B), jnp.float32),
    )(xT, gw, gw, gb_row, gb_row, lbl_row, ids_row, sf_row, tf_row)


def kernel(inputs, labels, softmax_w, softmax_b, sample_ids, true_freq,
           sample_freq):
    B = inputs.shape[0]
    S = sample_ids.shape[0]
    z1 = jnp.zeros((1,), jnp.int32)
    ids_all = jnp.concatenate([
        z1, sample_ids, jnp.zeros((_W1 - S - 1,), jnp.int32),
        labels, jnp.zeros((_NIDS - _W1 - B,), jnp.int32)])
    gw, gb = _sc_gather(softmax_w, softmax_b.reshape(-1), ids_all)

    neg1 = jnp.full((1,), -1, jnp.int32)
    ids_row = jnp.concatenate(
        [neg1, sample_ids, jnp.full((_W1 - S - 1,), -1, jnp.int32)]
    ).reshape(1, _W1)
    one1 = jnp.ones((1,), jnp.float32)
    sf_row = jnp.concatenate(
        [one1, sample_freq, jnp.ones((_W1 - S - 1,), jnp.float32)]
    ).reshape(1, _W1)

    logitsT = _tc_logits(inputs.T, gw, gb.reshape(1, _NIDS),
                         labels.reshape(1, B), ids_row, sf_row,
                         true_freq.reshape(1, B))
    return logitsT.T, jnp.zeros((B,), jnp.int32)
